# baseline (speedup 1.0000x reference)
import jax
import jax.numpy as jnp
from jax.experimental import pallas as pl

NUM_CLASSES = 1000
ROWS = 1024 * 50
BLOCK_ROWS = 512


def _onehot_block(x_ref, o_ref):
    idx = x_ref[...]
    iota = jax.lax.broadcasted_iota(jnp.int32, (BLOCK_ROWS, NUM_CLASSES), 1)
    o_ref[...] = (iota == idx).astype(jnp.int32)


def kernel(x):
    xf = x.reshape(ROWS, 1)
    out = pl.pallas_call(
        _onehot_block,
        grid=(ROWS // BLOCK_ROWS,),
        in_specs=[pl.BlockSpec((BLOCK_ROWS, 1), lambda i: (i, 0))],
        out_specs=pl.BlockSpec((BLOCK_ROWS, NUM_CLASSES), lambda i: (i, 0)),
        out_shape=jax.ShapeDtypeStruct((ROWS, NUM_CLASSES), jnp.int32),
    )(xf)
    return out.reshape(1024, 50, NUM_CLASSES)
